# baseline (speedup 1.0000x reference)
import functools

import jax
import jax.numpy as jnp
from jax import lax
from jax.experimental import pallas as pl
from jax.experimental.pallas import tpu as pltpu
from jax.experimental.pallas import tpu_sc as plsc

CG = 32
CX = 16
L = 16


def kernel(x, pos, pos_embedding):
    B, S, D = x.shape
    N = B * S
    xf = x.reshape(N, D)
    idx = pos.reshape(N).astype(jnp.int32)

    info = plsc.get_sparse_core_info()
    NC, NS = info.num_cores, info.num_subcores
    NW = NC * NS
    R = N // NW
    ng = R // CG
    nx = R // CX

    mesh = plsc.VectorSubcoreMesh(core_axis_name="core", subcore_axis_name="subcore")

    @functools.partial(
        pl.kernel,
        out_type=jax.ShapeDtypeStruct((N, D), x.dtype),
        mesh=mesh,
        scratch_types=[
            pltpu.VMEM((R,), jnp.int32),
            pltpu.VMEM((CG, D), jnp.float32),
            pltpu.VMEM((CG, D), jnp.float32),
            pltpu.VMEM((CX, D), jnp.float32),
            pltpu.VMEM((CX, D), jnp.float32),
            pltpu.SemaphoreType.DMA,
            pltpu.SemaphoreType.DMA,
            pltpu.SemaphoreType.DMA,
            pltpu.SemaphoreType.DMA,
            pltpu.SemaphoreType.DMA,
            pltpu.SemaphoreType.DMA,
        ],
    )
    def run(x_hbm, i_hbm, t_hbm, o_hbm, idx_v,
            bg0, bg1, bx0, bx1, sg0, sg1, sx0, sx1, so0, so1):
        bg = (bg0, bg1)
        bx = (bx0, bx1)
        sg = (sg0, sg1)
        sx = (sx0, sx1)
        so = (so0, so1)

        wid = lax.axis_index("core") * NS + lax.axis_index("subcore")
        base = wid * R
        pltpu.sync_copy(i_hbm.at[pl.ds(base, R)], idx_v)

        def start_g(g, b):
            pltpu.async_copy(t_hbm.at[idx_v.at[pl.ds(g * CG, CG)]], bg[b], sg[b])

        def wait_g(g, b):
            pltpu.make_async_copy(
                t_hbm.at[idx_v.at[pl.ds(g * CG, CG)]], bg[b], sg[b]).wait()

        def start_x(k, b):
            pltpu.async_copy(x_hbm.at[pl.ds(base + k * CX, CX)], bx[b], sx[b])

        def wait_x(k, b):
            pltpu.make_async_copy(
                x_hbm.at[pl.ds(base + k * CX, CX)], bx[b], sx[b]).wait()

        def wait_out(b):
            pltpu.make_async_copy(bg[b], o_hbm.at[pl.ds(base, CG)], so[b]).wait()

        start_g(0, 0)
        start_x(0, 0)

        @pl.loop(0, ng // 2)
        def _(p):
            for u in range(2):
                g = p * 2 + u

                @pl.when(g + 1 < ng)
                def _():
                    @pl.when(g + 1 >= 2)
                    def _():
                        wait_out(1 - u)

                    start_g(g + 1, 1 - u)

                wait_g(g, u)

                for h in range(CG // CX):
                    k = g * (CG // CX) + h

                    @pl.when(k + 1 < nx)
                    def _():
                        start_x(k + 1, 1 - h)

                    wait_x(k, h)

                    @pl.loop(0, CX)
                    def _(r):
                        row = h * CX + r
                        for c0 in range(0, D, 16 * L):
                            vals = [
                                bx[h].at[(r, pl.ds(c0 + j * L, L))][...]
                                for j in range(16)
                            ]
                            for j in range(16):
                                plsc.addupdate(
                                    bg[u].at[(row, pl.ds(c0 + j * L, L))],
                                    vals[j])

                pltpu.async_copy(bg[u], o_hbm.at[pl.ds(base + g * CG, CG)], so[u])

        wait_out(0)
        wait_out(1)

    out = run(xf, idx, pos_embedding)
    return out.reshape(B, S, D)


# speedup vs baseline: 1.1768x; 1.1768x over previous
import functools

import jax
import jax.numpy as jnp
from jax import lax
from jax.experimental import pallas as pl
from jax.experimental.pallas import tpu as pltpu
from jax.experimental.pallas import tpu_sc as plsc

C = 16
L = 16
NB = 4
U = 4


def kernel(x, pos, pos_embedding):
    B, S, D = x.shape
    N = B * S
    xf = x.reshape(N, D)
    idx = pos.reshape(N).astype(jnp.int32)

    info = plsc.get_sparse_core_info()
    NC, NS = info.num_cores, info.num_subcores
    NW = NC * NS
    R = N // NW
    ng = R // C

    mesh = plsc.VectorSubcoreMesh(core_axis_name="core", subcore_axis_name="subcore")

    @functools.partial(
        pl.kernel,
        out_type=jax.ShapeDtypeStruct((N, D), x.dtype),
        mesh=mesh,
        scratch_types=[
            pltpu.VMEM((R,), jnp.int32),
            pltpu.VMEM((C, D), jnp.float32),
            pltpu.VMEM((C, D), jnp.float32),
            pltpu.VMEM((C, D), jnp.float32),
            pltpu.VMEM((C, D), jnp.float32),
            pltpu.VMEM((C, D), jnp.float32),
            pltpu.VMEM((C, D), jnp.float32),
            pltpu.SemaphoreType.DMA,
            pltpu.SemaphoreType.DMA,
            pltpu.SemaphoreType.DMA,
            pltpu.SemaphoreType.DMA,
            pltpu.SemaphoreType.DMA,
            pltpu.SemaphoreType.DMA,
            pltpu.SemaphoreType.DMA,
            pltpu.SemaphoreType.DMA,
            pltpu.SemaphoreType.DMA,
            pltpu.SemaphoreType.DMA,
        ],
    )
    def run(x_hbm, i_hbm, t_hbm, o_hbm, idx_v,
            bg0, bg1, bg2, bg3, bx0, bx1,
            sg0, sg1, sg2, sg3, sx0, sx1, so0, so1, so2, so3):
        bg = (bg0, bg1, bg2, bg3)
        bx = (bx0, bx1)
        sg = (sg0, sg1, sg2, sg3)
        sx = (sx0, sx1)
        so = (so0, so1, so2, so3)

        wid = lax.axis_index("core") * NS + lax.axis_index("subcore")
        base = wid * R
        pltpu.sync_copy(i_hbm.at[pl.ds(base, R)], idx_v)

        def start_g(g, b):
            pltpu.async_copy(t_hbm.at[idx_v.at[pl.ds(g * C, C)]], bg[b], sg[b])

        def wait_g(g, b):
            pltpu.make_async_copy(
                t_hbm.at[idx_v.at[pl.ds(g * C, C)]], bg[b], sg[b]).wait()

        def start_x(g, b):
            pltpu.async_copy(x_hbm.at[pl.ds(base + g * C, C)], bx[b], sx[b])

        def wait_x(g, b):
            pltpu.make_async_copy(
                x_hbm.at[pl.ds(base + g * C, C)], bx[b], sx[b]).wait()

        def wait_out(b):
            pltpu.make_async_copy(bg[b], o_hbm.at[pl.ds(base, C)], so[b]).wait()

        start_g(0, 0)
        start_g(1, 1)
        start_x(0, 0)

        @pl.loop(0, ng // U)
        def _(p):
            for u in range(U):
                g = p * U + u

                @pl.when(g + 2 < ng)
                def _():
                    @pl.when(g >= 2)
                    def _():
                        wait_out((u + 2) % NB)

                    start_g(g + 2, (u + 2) % NB)

                @pl.when(g + 1 < ng)
                def _():
                    start_x(g + 1, (u + 1) % 2)

                wait_g(g, u % NB)
                wait_x(g, u % 2)

                @pl.loop(0, C)
                def _(r):
                    for c0 in range(0, D, 8 * L):
                        vals = [
                            bx[u % 2].at[(r, pl.ds(c0 + j * L, L))][...]
                            for j in range(8)
                        ]
                        for j in range(8):
                            plsc.addupdate(
                                bg[u % NB].at[(r, pl.ds(c0 + j * L, L))],
                                vals[j])

                pltpu.async_copy(
                    bg[u % NB], o_hbm.at[pl.ds(base + g * C, C)], so[u % NB])

        for b in range(NB):
            wait_out(b)

    out = run(xf, idx, pos_embedding)
    return out.reshape(B, S, D)
